# baseline (speedup 1.0000x reference)
import functools

import jax
import jax.numpy as jnp
from jax import lax
from jax.experimental import pallas as pl
from jax.experimental.pallas import tpu as pltpu
from jax.experimental.pallas import tpu_sc as plsc

_B, _S, _D = 4, 4096, 1024
_N = _B * _S
_NC, _NS = 2, 16
_NW = _NC * _NS
_ROWS_W = _N // _NW
_CH = 32
_NCHUNK = _ROWS_W // _CH
_UNROLL = 8
_L = 16


def _sc_add(x_hbm, pe_hbm, o_hbm, xbuf, pebuf):
    c = lax.axis_index("c")
    s = lax.axis_index("s")
    wid = s * _NC + c
    base = wid * _ROWS_W
    pe_base = lax.rem(base, _S)

    def chunk(i, carry):
        off = (base + i * _CH) * _D
        pe_off = (pe_base + i * _CH) * _D
        pltpu.sync_copy(x_hbm.at[pl.ds(off, _CH * _D)], xbuf)
        pltpu.sync_copy(pe_hbm.at[pl.ds(pe_off, _CH * _D)], pebuf)

        def vec(j, carry2):
            b = j * (_L * _UNROLL)
            for u in range(_UNROLL):
                sl = pl.ds(b + u * _L, _L)
                xbuf[sl] = xbuf[sl] + pebuf[sl]
            return carry2

        lax.fori_loop(0, _CH * _D // (_L * _UNROLL), vec, 0)
        pltpu.sync_copy(xbuf, o_hbm.at[pl.ds(off, _CH * _D)])
        return carry

    lax.fori_loop(0, _NCHUNK, chunk, 0)


@functools.partial(
    pl.kernel,
    out_type=jax.ShapeDtypeStruct((_N * _D,), jnp.float32),
    mesh=plsc.VectorSubcoreMesh(core_axis_name="c", subcore_axis_name="s"),
    scratch_types=[
        pltpu.VMEM((_CH * _D,), jnp.float32),
        pltpu.VMEM((_CH * _D,), jnp.float32),
    ],
)
def _sc_kernel(x_hbm, pe_hbm, o_hbm, xbuf, pebuf):
    _sc_add(x_hbm, pe_hbm, o_hbm, xbuf, pebuf)


def kernel(x, pe_table):
    B, S, D = x.shape
    x_flat = x.reshape(-1)
    pe_flat = pe_table[:S].reshape(-1)
    out_flat = _sc_kernel(x_flat, pe_flat)
    return out_flat.reshape(B, S, D)


# speedup vs baseline: 1.2106x; 1.2106x over previous
import functools

import jax
import jax.numpy as jnp
from jax import lax
from jax.experimental import pallas as pl
from jax.experimental.pallas import tpu as pltpu
from jax.experimental.pallas import tpu_sc as plsc

_B, _S, _D = 4, 4096, 1024
_N = _B * _S
_NC, _NS = 2, 16
_NW = _NC * _NS
_ROWS_W = _N // _NW
_CH = 16
_NCHUNK = _ROWS_W // _CH
_NBUF = 3
_UNROLL = 16
_L = 16
_CD = _CH * _D


def _sc_body(x_hbm, pe_hbm, o_hbm, xbufs, pebufs, xsems, psems, osems):
    c = lax.axis_index("c")
    s = lax.axis_index("s")
    wid = s * _NC + c
    base = wid * _ROWS_W
    pe_base = lax.rem(base, _S)

    def in_copies(i):
        sl = i % _NBUF
        off = (base + i * _CH) * _D
        pe_off = (pe_base + i * _CH) * _D
        return (
            pltpu.make_async_copy(x_hbm.at[pl.ds(off, _CD)], xbufs[sl], xsems[sl]),
            pltpu.make_async_copy(pe_hbm.at[pl.ds(pe_off, _CD)], pebufs[sl], psems[sl]),
        )

    def out_copy(i):
        sl = i % _NBUF
        off = (base + i * _CH) * _D
        return pltpu.make_async_copy(xbufs[sl], o_hbm.at[pl.ds(off, _CD)], osems[sl])

    for cp in in_copies(0) + in_copies(1):
        cp.start()

    for i in range(_NCHUNK):
        sl = i % _NBUF
        for cp in in_copies(i):
            cp.wait()
        if i + 2 < _NCHUNK:
            if i >= 1:
                out_copy(i - 1).wait()
            for cp in in_copies(i + 2):
                cp.start()

        xv = xbufs[sl]
        pv = pebufs[sl]

        def vec(j, carry):
            b = j * (_L * _UNROLL)
            for u in range(_UNROLL):
                ds = pl.ds(b + u * _L, _L)
                xv[ds] = xv[ds] + pv[ds]
            return carry

        lax.fori_loop(0, _CD // (_L * _UNROLL), vec, 0)
        out_copy(i).start()

    for i in range(_NCHUNK - 3, _NCHUNK):
        out_copy(i).wait()


@functools.partial(
    pl.kernel,
    out_type=jax.ShapeDtypeStruct((_N * _D,), jnp.float32),
    mesh=plsc.VectorSubcoreMesh(core_axis_name="c", subcore_axis_name="s"),
    scratch_types=[
        [pltpu.VMEM((_CD,), jnp.float32)] * _NBUF,
        [pltpu.VMEM((_CD,), jnp.float32)] * _NBUF,
        [pltpu.SemaphoreType.DMA] * _NBUF,
        [pltpu.SemaphoreType.DMA] * _NBUF,
        [pltpu.SemaphoreType.DMA] * _NBUF,
    ],
)
def _sc_kernel(x_hbm, pe_hbm, o_hbm, xbufs, pebufs, xsems, psems, osems):
    _sc_body(x_hbm, pe_hbm, o_hbm, xbufs, pebufs, xsems, psems, osems)


def kernel(x, pe_table):
    B, S, D = x.shape
    x_flat = x.reshape(-1)
    pe_flat = pe_table[:S].reshape(-1)
    out_flat = _sc_kernel(x_flat, pe_flat)
    return out_flat.reshape(B, S, D)


# speedup vs baseline: 5.9274x; 4.8962x over previous
import jax
import jax.numpy as jnp
from jax.experimental import pallas as pl
from jax.experimental.pallas import tpu as pltpu

_BS = 2048


def _add_body(x_ref, pe_ref, o_ref):
    o_ref[...] = x_ref[...] + pe_ref[...][None]


def kernel(x, pe_table):
    B, S, D = x.shape
    grid = (S // _BS, B)
    return pl.pallas_call(
        _add_body,
        grid=grid,
        in_specs=[
            pl.BlockSpec((1, _BS, D), lambda s, b: (b, s, 0)),
            pl.BlockSpec((_BS, D), lambda s, b: (s, 0)),
        ],
        out_specs=pl.BlockSpec((1, _BS, D), lambda s, b: (b, s, 0)),
        out_shape=jax.ShapeDtypeStruct((B, S, D), x.dtype),
        compiler_params=pltpu.CompilerParams(
            dimension_semantics=("arbitrary", "arbitrary"),
        ),
    )(x, pe_table)
